# baseline (speedup 1.0000x reference)
import functools

import jax
import jax.numpy as jnp
from jax import lax
from jax.experimental import pallas as pl
from jax.experimental.pallas import tpu as pltpu
from jax.experimental.pallas import tpu_sc as plsc


def kernel(tokenized_prompts, ctx, token_embedding):
    n_cls, seq = tokenized_prompts.shape
    n_ctx, d = ctx.shape
    vocab = token_embedding.shape[0]

    table = jnp.concatenate([token_embedding, ctx], axis=0)
    pos = jnp.arange(seq, dtype=jnp.int32)[None, :]
    ctx_ids = (vocab - 1 + pos).astype(jnp.int32)
    idx = jnp.where((pos >= 1) & (pos < 1 + n_ctx), ctx_ids,
                    tokenized_prompts)
    idx = jnp.pad(idx, ((0, 0), (0, 128 - seq))).reshape(-1)

    info = plsc.get_sparse_core_info()
    nc, ns = info.num_cores, info.num_subcores
    nw = nc * ns
    per_w = n_cls // nw
    nbuf = 6
    chunk = 32
    n_ch = per_w // chunk
    n_u = 2 * chunk
    h0 = 40
    h1 = seq - h0

    mesh = plsc.VectorSubcoreMesh(core_axis_name="c", subcore_axis_name="s")

    @functools.partial(
        pl.kernel,
        out_type=jax.ShapeDtypeStruct((n_cls, seq, d), jnp.float32),
        mesh=mesh,
        scratch_types=[
            pltpu.VMEM((chunk * 128,), jnp.int32),
            pltpu.VMEM((nbuf, h0, d), jnp.float32),
            [pltpu.SemaphoreType.DMA] * nbuf,
            [pltpu.SemaphoreType.DMA] * nbuf,
        ],
    )
    def _gather_kernel(idx_hbm, table_hbm, out_hbm, idx_v, rows_v,
                       gsems, ssems):
        wid = lax.axis_index("s") * nc + lax.axis_index("c")
        base = wid * per_w

        @pl.loop(0, n_ch)
        def _outer(j):
            cbase = base + j * chunk
            pltpu.sync_copy(idx_hbm.at[pl.ds(cbase * 128, chunk * 128)],
                            idx_v)
            tail = j * 0 + 72

            def gather_desc(u, b, h):
                c = u // 2
                src = table_hbm.at[idx_v.at[pl.ds(c * 128 + h * h0, h0)]]
                return pltpu.make_async_copy(src, rows_v.at[b], gsems[b])

            def store_desc(u, b, h):
                c = u // 2
                row = out_hbm.at[cbase + c]
                if h == 0:
                    return (
                        pltpu.make_async_copy(
                            rows_v.at[b], row.at[pl.ds(0, h0)], ssems[b]),
                    )
                return (
                    pltpu.make_async_copy(
                        rows_v.at[b, pl.ds(0, 32)], row.at[pl.ds(h0, 32)],
                        ssems[b]),
                    pltpu.make_async_copy(
                        rows_v.at[b, pl.ds(32, 8)], row.at[pl.ds(tail, 8)],
                        ssems[b]),
                )

            gather_desc(0, 0, 0).start()
            gather_desc(1, 1, 1).start()

            @pl.loop(0, n_u, step=nbuf)
            def _body(n):
                for b in range(nbuf):
                    u = n + b
                    bn = (b + 2) % nbuf

                    @pl.when(u < n_u)
                    def _():
                        gather_desc(u, b, b % 2).wait()
                        for dsc in store_desc(u, b, b % 2):
                            dsc.start()

                    @pl.when(u >= 4)
                    def _():
                        for dsc in store_desc(u - 4, bn, b % 2):
                            dsc.wait()

                    @pl.when(u + 2 < n_u)
                    def _():
                        gather_desc(u + 2, bn, b % 2).start()

            last_n = nbuf * ((n_u - 1) // nbuf)
            u_max = last_n + nbuf - 1
            drained = u_max - 4
            for u in range(max(0, drained + 1), n_u):
                for dsc in store_desc(u, u % nbuf, u % 2):
                    dsc.wait()

    return _gather_kernel(idx, table)


# speedup vs baseline: 1.1959x; 1.1959x over previous
import functools

import jax
import jax.numpy as jnp
from jax import lax
from jax.experimental import pallas as pl
from jax.experimental.pallas import tpu as pltpu
from jax.experimental.pallas import tpu_sc as plsc


def kernel(tokenized_prompts, ctx, token_embedding):
    n_cls, seq = tokenized_prompts.shape
    n_ctx, d = ctx.shape
    n_suf = seq - 1 - n_ctx
    ng = 1 + n_suf

    tok_c = jnp.concatenate(
        [tokenized_prompts[:, :1], tokenized_prompts[:, 1 + n_ctx:]], axis=1)

    info = plsc.get_sparse_core_info()
    nc, ns = info.num_cores, info.num_subcores
    nw = nc * ns
    per_w = n_cls // nw
    nbuf = 3

    mesh = plsc.VectorSubcoreMesh(core_axis_name="c", subcore_axis_name="s")

    @functools.partial(
        pl.kernel,
        out_type=jax.ShapeDtypeStruct((n_cls, seq, d), jnp.float32),
        mesh=mesh,
        scratch_types=[
            pltpu.VMEM((per_w, ng), jnp.int32),
            pltpu.VMEM((nbuf, ng, d), jnp.float32),
            pltpu.VMEM((n_ctx, d), jnp.float32),
            [pltpu.SemaphoreType.DMA] * nbuf,
            [pltpu.SemaphoreType.DMA] * nbuf,
        ],
        compiler_params=pltpu.CompilerParams(use_tc_tiling_on_sc=False),
    )
    def _gather_kernel(tok_hbm, ctx_hbm, table_hbm, out_hbm,
                       idx_v, rows_v, ctx_v, gsems, ssems):
        wid = lax.axis_index("s") * nc + lax.axis_index("c")
        base = wid * per_w

        pltpu.sync_copy(ctx_hbm, ctx_v)
        pltpu.sync_copy(tok_hbm.at[pl.ds(base, per_w)], idx_v)

        def issue_gather(k, b):
            pltpu.async_copy(
                table_hbm.at[idx_v.at[k]], rows_v.at[b], gsems[b])

        def store_descs(k, b):
            row = out_hbm.at[base + k]
            return (
                (rows_v.at[b, pl.ds(0, 1)], row.at[pl.ds(0, 1)]),
                (ctx_v, row.at[pl.ds(1, n_ctx)]),
                (rows_v.at[b, pl.ds(1, n_suf)], row.at[pl.ds(1 + n_ctx, n_suf)]),
            )

        issue_gather(0, 0)

        @pl.loop(0, per_w, step=nbuf)
        def _body(n):
            for b in range(nbuf):
                k = n + b
                bn = (b + 1) % nbuf

                @pl.when(k < per_w)
                def _():
                    pltpu.make_async_copy(
                        table_hbm.at[idx_v.at[k]], rows_v.at[b], gsems[b]
                    ).wait()
                    for src, dst in store_descs(k, b):
                        pltpu.async_copy(src, dst, ssems[b])

                @pl.when(k >= 2)
                def _():
                    for src, dst in store_descs(k - 2, bn):
                        pltpu.make_async_copy(src, dst, ssems[bn]).wait()

                @pl.when(k + 1 < per_w)
                def _():
                    issue_gather(k + 1, bn)

        for k in range(max(0, per_w - 1), per_w):
            b = k % nbuf
            for src, dst in store_descs(k, b):
                pltpu.make_async_copy(src, dst, ssems[b]).wait()

    return _gather_kernel(tok_c, ctx, token_embedding)
